# baseline (speedup 1.0000x reference)
import jax
import jax.numpy as jnp
from jax.experimental import pallas as pl
from jax.experimental.pallas import tpu as pltpu

N_ROWS = 16384
D = 4096
EMA_M = 0.05
BR = 256
GRID = N_ROWS // BR


def _body(slot_ref, z_ref, p_ref, out_ref, acc_ref):
    i = pl.program_id(0)

    @pl.when(i == 0)
    def _init():
        acc_ref[...] = jnp.zeros_like(acc_ref)

    acc_ref[...] += jnp.sum(z_ref[...].reshape(BR // 8, 8, D), axis=0)

    @pl.when(i == GRID - 1)
    def _finish():
        out_ref[...] = p_ref[...]
        zmean = jnp.sum(acc_ref[...], axis=0, keepdims=True) * (1.0 / N_ROWS)
        slot = slot_ref[0]
        old = p_ref[pl.ds(slot, 1), :]
        out_ref[pl.ds(slot, 1), :] = (1.0 - EMA_M) * old + EMA_M * zmean


def kernel(z, P_tumor_main, m_idx, r_idx):
    M, R, K, Dd = P_tumor_main.shape
    p2 = P_tumor_main.reshape(M * R * K, Dd)
    slot = (jnp.asarray(m_idx, jnp.int32) * R + jnp.asarray(r_idx, jnp.int32)).reshape(1)
    out = pl.pallas_call(
        _body,
        grid_spec=pltpu.PrefetchScalarGridSpec(
            num_scalar_prefetch=1,
            grid=(GRID,),
            in_specs=[
                pl.BlockSpec((BR, D), lambda i, slot_ref: (i, 0)),
                pl.BlockSpec((M * R * K, Dd), lambda i, slot_ref: (0, 0)),
            ],
            out_specs=pl.BlockSpec((M * R * K, Dd), lambda i, slot_ref: (0, 0)),
            scratch_shapes=[pltpu.VMEM((8, D), jnp.float32)],
        ),
        out_shape=jax.ShapeDtypeStruct((M * R * K, Dd), jnp.float32),
        compiler_params=pltpu.CompilerParams(
            dimension_semantics=("arbitrary",),
        ),
    )(slot, z, p2)
    return out.reshape(M, R, K, Dd)


# speedup vs baseline: 1.0723x; 1.0723x over previous
import jax
import jax.numpy as jnp
from jax.experimental import pallas as pl
from jax.experimental.pallas import tpu as pltpu

N_ROWS = 16384
D = 4096
EMA_M = 0.05
BR = 1024
GRID = N_ROWS // BR


def _body(idx_ref, z_ref, p_ref, out_ref, acc_ref):
    i = pl.program_id(0)

    @pl.when(i == 0)
    def _init():
        acc_ref[...] = jnp.zeros_like(acc_ref)

    acc_ref[...] += jnp.sum(z_ref[...].reshape(BR // 8, 8, D), axis=0)

    @pl.when(i == GRID - 1)
    def _finish():
        out_ref[...] = p_ref[...]
        zmean = jnp.sum(acc_ref[...], axis=0, keepdims=True) * (1.0 / N_ROWS)
        m, r = idx_ref[0], idx_ref[1]
        old = p_ref[pl.ds(m, 1), pl.ds(r, 1), 0, :]
        new = (1.0 - EMA_M) * old + EMA_M * zmean.reshape(1, 1, D)
        out_ref[pl.ds(m, 1), pl.ds(r, 1), 0, :] = new


def kernel(z, P_tumor_main, m_idx, r_idx):
    M, R, K, Dd = P_tumor_main.shape
    idx = jnp.stack(
        [jnp.asarray(m_idx, jnp.int32), jnp.asarray(r_idx, jnp.int32)]
    )
    return pl.pallas_call(
        _body,
        grid_spec=pltpu.PrefetchScalarGridSpec(
            num_scalar_prefetch=1,
            grid=(GRID,),
            in_specs=[
                pl.BlockSpec((BR, D), lambda i, idx_ref: (i, 0)),
                pl.BlockSpec((M, R, K, Dd), lambda i, idx_ref: (0, 0, 0, 0)),
            ],
            out_specs=pl.BlockSpec((M, R, K, Dd), lambda i, idx_ref: (0, 0, 0, 0)),
            scratch_shapes=[pltpu.VMEM((8, D), jnp.float32)],
        ),
        out_shape=jax.ShapeDtypeStruct((M, R, K, Dd), jnp.float32),
        compiler_params=pltpu.CompilerParams(
            dimension_semantics=("arbitrary",),
        ),
    )(idx, z, P_tumor_main)


# speedup vs baseline: 1.0888x; 1.0154x over previous
import jax
import jax.numpy as jnp
from jax.experimental import pallas as pl
from jax.experimental.pallas import tpu as pltpu

N_ROWS = 16384
D = 4096
EMA_M = 0.05
BR = 1024
GRID = N_ROWS // BR


def _body(m_ref, r_ref, z_ref, p_ref, out_ref, acc_ref):
    i = pl.program_id(0)

    @pl.when(i == 0)
    def _init():
        acc_ref[...] = jnp.zeros_like(acc_ref)

    acc_ref[...] += jnp.sum(z_ref[...].reshape(BR // 8, 8, D), axis=0)

    @pl.when(i == GRID - 1)
    def _finish():
        out_ref[...] = p_ref[...]
        zmean = jnp.sum(acc_ref[...], axis=0, keepdims=True) * (1.0 / N_ROWS)
        m, r = m_ref[0], r_ref[0]
        old = p_ref[pl.ds(m, 1), pl.ds(r, 1), 0, :]
        new = (1.0 - EMA_M) * old + EMA_M * zmean.reshape(1, 1, D)
        out_ref[pl.ds(m, 1), pl.ds(r, 1), 0, :] = new


def kernel(z, P_tumor_main, m_idx, r_idx):
    M, R, K, Dd = P_tumor_main.shape
    m1 = jnp.asarray(m_idx, jnp.int32).reshape(1)
    r1 = jnp.asarray(r_idx, jnp.int32).reshape(1)
    return pl.pallas_call(
        _body,
        grid_spec=pltpu.PrefetchScalarGridSpec(
            num_scalar_prefetch=2,
            grid=(GRID,),
            in_specs=[
                pl.BlockSpec((BR, D), lambda i, m, r: (i, 0)),
                pl.BlockSpec((M, R, K, Dd), lambda i, m, r: (0, 0, 0, 0)),
            ],
            out_specs=pl.BlockSpec((M, R, K, Dd), lambda i, m, r: (0, 0, 0, 0)),
            scratch_shapes=[pltpu.VMEM((8, D), jnp.float32)],
        ),
        out_shape=jax.ShapeDtypeStruct((M, R, K, Dd), jnp.float32),
        compiler_params=pltpu.CompilerParams(
            dimension_semantics=("arbitrary",),
        ),
    )(m1, r1, z, P_tumor_main)
